# baseline (speedup 1.0000x reference)
import functools

import jax
import jax.numpy as jnp
from jax import lax
from jax.experimental import pallas as pl
from jax.experimental.pallas import tpu as pltpu

_AGENT_OFF = 1
_ACTION_OFF = 11
_LOC_OFF = 33
_NA, _NAC, _NL = 10, 22, 16
_NAC_P = 32
_NIJ = _NA * _NAC_P


def _loss_kernel(p_ref, inv_d_ref, inv_t_ref, out_ref,
                 vad_ref, vbd_ref, vat_ref, vbt_ref, vct_ref, u22_ref,
                 *, inv_nd, inv_nt, nv_d, nv_t):
    if True:
        nd = inv_d_ref.shape[0]
        nt = inv_t_ref.shape[0]
        f32 = jnp.float32
        col = lax.broadcasted_iota(jnp.int32, (1, _NIJ), 1)
        u_row = (col % _NAC_P < _NAC).astype(f32)
        u22_ref[...] = (lax.broadcasted_iota(jnp.int32, (1, _NAC_P), 1)
                        < _NAC).astype(jnp.bfloat16)
        i10 = lax.broadcasted_iota(jnp.int32, (1, _NA), 1).astype(f32)
        j32 = lax.broadcasted_iota(jnp.int32, (1, _NAC_P), 1).astype(f32)
        k16 = lax.broadcasted_iota(jnp.int32, (1, _NL), 1).astype(f32)
        dij = inv_d_ref[:, 0:1] * _NAC_P + inv_d_ref[:, 1:2]
        e_d = (dij == lax.broadcasted_iota(jnp.int32, (nd, _NIJ), 1)
               ).astype(f32)
        wd = jnp.dot(jnp.full((1, nd), 1.0, f32), e_d,
                     preferred_element_type=f32)
        flat_d = lax.broadcasted_iota(jnp.int32, (1, _NIJ), 1).astype(f32)
        score = (u_row - wd) * (flat_d + 1.0)
        for t in range(nv_d):
            pos = jnp.max(score) - 1.0
            ii = jnp.floor((pos + 0.5) / _NAC_P)
            jj = pos - ii * _NAC_P
            vad_ref[t:t + 1, :] = (i10 == ii).astype(jnp.bfloat16)
            vbd_ref[t:t + 1, :] = (j32 == jj).astype(jnp.bfloat16)
            score = score * (1.0 - (flat_d == pos).astype(f32))
        tij = inv_t_ref[:, 0:1] * _NAC_P + inv_t_ref[:, 1:2]
        e_ij = (tij == lax.broadcasted_iota(jnp.int32, (nt, _NIJ), 1)
                ).astype(f32)
        ekT = (lax.broadcasted_iota(jnp.int32, (_NL, nt), 0)
               == inv_t_ref[:, 2:3].T).astype(f32)
        wt = jnp.dot(ekT, e_ij, preferred_element_type=f32)
        flat_t = (lax.broadcasted_iota(jnp.int32, (_NL, _NIJ), 0) * _NIJ
                  + lax.broadcasted_iota(jnp.int32, (_NL, _NIJ), 1)
                  ).astype(f32)
        score_t = (jnp.broadcast_to(u_row, (_NL, _NIJ)) - wt) * (flat_t + 1.0)
        for t in range(nv_t):
            pos = jnp.max(score_t) - 1.0
            kk = jnp.floor((pos + 0.5) / _NIJ)
            ij = pos - kk * _NIJ
            ii = jnp.floor((ij + 0.5) / _NAC_P)
            jj = ij - ii * _NAC_P
            vat_ref[t:t + 1, :] = (i10 == ii).astype(jnp.bfloat16)
            vbt_ref[t:t + 1, :] = (j32 == jj).astype(jnp.bfloat16)
            vct_ref[t:t + 1, :] = (k16 == kk).astype(jnp.bfloat16)
            score_t = score_t * (1.0 - (flat_t == pos).astype(f32))

    p = p_ref[...]
    a = p[_AGENT_OFF:_AGENT_OFF + _NA, :]
    b = p[_ACTION_OFF:_ACTION_OFF + _NAC_P, :]
    c = p[_LOC_OFF:_LOC_OFF + _NL, :]
    fa = jnp.minimum(a, c[0:1, :])
    fb = jnp.minimum(b, c[0:1, :])
    for k in range(1, _NL):
        ck = c[k:k + 1, :]
        fa += jnp.minimum(a, ck)
        fb += jnp.minimum(b, ck)
    accd = jnp.minimum(b, a[0:1, :])
    acct = jnp.minimum(fb, fa[0:1, :])
    for i in range(1, _NA):
        accd += jnp.minimum(b, a[i:i + 1, :])
        acct += jnp.minimum(fb, fa[i:i + 1, :])
    u22 = u22_ref[...]
    dup = jnp.dot(u22, accd, preferred_element_type=jnp.float32)
    trip = jnp.dot(u22, acct, preferred_element_type=jnp.float32)
    if nv_d:
        ad = jnp.dot(vad_ref[...], a, preferred_element_type=jnp.float32)
        bd = jnp.dot(vbd_ref[...], b, preferred_element_type=jnp.float32)
        dup -= jnp.sum(jnp.minimum(ad, bd), axis=0, keepdims=True)
    if nv_t:
        at = jnp.dot(vat_ref[...], a, preferred_element_type=jnp.float32)
        bt = jnp.dot(vbt_ref[...], b, preferred_element_type=jnp.float32)
        ct = jnp.dot(vct_ref[...], c, preferred_element_type=jnp.float32)
        trip -= jnp.sum(jnp.minimum(jnp.minimum(at, bt), ct),
                        axis=0, keepdims=True)
    part = jnp.sum(dup * inv_nd + trip * inv_nt, keepdims=True)
    out_ref[...] = part.reshape(1, 1, 1)


def kernel(preds, inv_d, inv_t):
    preds16 = preds.T.astype(jnp.bfloat16)
    inv_d = inv_d.astype(jnp.int32)
    inv_t = inv_t.astype(jnp.int32)
    n, ncols = preds.shape
    nd, nt = inv_d.shape[0], inv_t.shape[0]
    nv_d = _NA * _NAC - nd
    nv_t = _NA * _NAC * _NL - nt

    blk = 8192
    while n % blk:
        blk //= 2
    nsteps = n // blk
    partials = pl.pallas_call(
        functools.partial(_loss_kernel, inv_nd=1.0 / (n * nd),
                          inv_nt=1.0 / (n * nt), nv_d=nv_d, nv_t=nv_t),
        grid=(nsteps,),
        in_specs=[
            pl.BlockSpec((ncols, blk), lambda s: (0, s)),
            pl.BlockSpec(inv_d.shape, lambda s: (0, 0)),
            pl.BlockSpec(inv_t.shape, lambda s: (0, 0)),
        ],
        out_specs=pl.BlockSpec((1, 1, 1), lambda s: (s, 0, 0)),
        out_shape=jax.ShapeDtypeStruct((nsteps, 1, 1), jnp.float32),
        compiler_params=pltpu.CompilerParams(
            dimension_semantics=("parallel",)),
        scratch_shapes=[pltpu.VMEM((max(nv_d, 1), _NA), jnp.bfloat16),
                        pltpu.VMEM((max(nv_d, 1), _NAC_P), jnp.bfloat16),
                        pltpu.VMEM((max(nv_t, 1), _NA), jnp.bfloat16),
                        pltpu.VMEM((max(nv_t, 1), _NAC_P), jnp.bfloat16),
                        pltpu.VMEM((max(nv_t, 1), _NL), jnp.bfloat16),
                        pltpu.VMEM((1, _NAC_P), jnp.bfloat16)],
    )(preds16, inv_d, inv_t)
    return jnp.sum(partials, axis=(0, 1))


# speedup vs baseline: 1.2159x; 1.2159x over previous
import functools

import jax
import jax.numpy as jnp
from jax import lax
from jax.experimental import pallas as pl
from jax.experimental.pallas import tpu as pltpu

_AGENT_OFF = 1
_ACTION_OFF = 11
_LOC_OFF = 33
_NA, _NAC, _NL = 10, 22, 16
_NAC_P = 32
_NIJ = _NA * _NAC_P


def _loss_kernel(p_ref, inv_d_ref, inv_t_ref, out_ref,
                 vad_ref, vbd_ref, vat_ref, vbt_ref, vct_ref, u22_ref,
                 *, inv_nd, inv_nt, nv_d, nv_t):
    s = pl.program_id(0)

    @pl.when(s == 0)
    def _build_selectors():
        nd = inv_d_ref.shape[0]
        nt = inv_t_ref.shape[0]
        f32 = jnp.float32
        col = lax.broadcasted_iota(jnp.int32, (1, _NIJ), 1)
        u_row = (col % _NAC_P < _NAC).astype(f32)
        u22_ref[...] = (lax.broadcasted_iota(jnp.int32, (1, _NAC_P), 1)
                        < _NAC).astype(jnp.bfloat16)
        i10 = lax.broadcasted_iota(jnp.int32, (1, _NA), 1).astype(f32)
        j32 = lax.broadcasted_iota(jnp.int32, (1, _NAC_P), 1).astype(f32)
        k16 = lax.broadcasted_iota(jnp.int32, (1, _NL), 1).astype(f32)
        dij = inv_d_ref[:, 0:1] * _NAC_P + inv_d_ref[:, 1:2]
        e_d = (dij == lax.broadcasted_iota(jnp.int32, (nd, _NIJ), 1)
               ).astype(f32)
        wd = jnp.dot(jnp.full((1, nd), 1.0, f32), e_d,
                     preferred_element_type=f32)
        flat_d = lax.broadcasted_iota(jnp.int32, (1, _NIJ), 1).astype(f32)
        score = (u_row - wd) * (flat_d + 1.0)
        for t in range(nv_d):
            pos = jnp.max(score) - 1.0
            ii = jnp.floor((pos + 0.5) / _NAC_P)
            jj = pos - ii * _NAC_P
            vad_ref[t:t + 1, :] = (i10 == ii).astype(jnp.bfloat16)
            vbd_ref[t:t + 1, :] = (j32 == jj).astype(jnp.bfloat16)
            score = score * (1.0 - (flat_d == pos).astype(f32))
        tij = inv_t_ref[:, 0:1] * _NAC_P + inv_t_ref[:, 1:2]
        e_ij = (tij == lax.broadcasted_iota(jnp.int32, (nt, _NIJ), 1)
                ).astype(f32)
        ekT = (lax.broadcasted_iota(jnp.int32, (_NL, nt), 0)
               == inv_t_ref[:, 2:3].T).astype(f32)
        wt = jnp.dot(ekT, e_ij, preferred_element_type=f32)
        flat_t = (lax.broadcasted_iota(jnp.int32, (_NL, _NIJ), 0) * _NIJ
                  + lax.broadcasted_iota(jnp.int32, (_NL, _NIJ), 1)
                  ).astype(f32)
        score_t = (jnp.broadcast_to(u_row, (_NL, _NIJ)) - wt) * (flat_t + 1.0)
        for t in range(nv_t):
            pos = jnp.max(score_t) - 1.0
            kk = jnp.floor((pos + 0.5) / _NIJ)
            ij = pos - kk * _NIJ
            ii = jnp.floor((ij + 0.5) / _NAC_P)
            jj = ij - ii * _NAC_P
            vat_ref[t:t + 1, :] = (i10 == ii).astype(jnp.bfloat16)
            vbt_ref[t:t + 1, :] = (j32 == jj).astype(jnp.bfloat16)
            vct_ref[t:t + 1, :] = (k16 == kk).astype(jnp.bfloat16)
            score_t = score_t * (1.0 - (flat_t == pos).astype(f32))
        out_ref[...] = jnp.zeros((1, 1), jnp.float32)

    p = p_ref[...]
    a = p[_AGENT_OFF:_AGENT_OFF + _NA, :]
    b = p[_ACTION_OFF:_ACTION_OFF + _NAC_P, :]
    c = p[_LOC_OFF:_LOC_OFF + _NL, :]
    fa = jnp.minimum(a, c[0:1, :])
    fb = jnp.minimum(b, c[0:1, :])
    for k in range(1, _NL):
        ck = c[k:k + 1, :]
        fa += jnp.minimum(a, ck)
        fb += jnp.minimum(b, ck)
    accd = jnp.minimum(b, a[0:1, :])
    acct = jnp.minimum(fb, fa[0:1, :])
    for i in range(1, _NA):
        accd += jnp.minimum(b, a[i:i + 1, :])
        acct += jnp.minimum(fb, fa[i:i + 1, :])
    u22 = u22_ref[...]
    dup = jnp.dot(u22, accd, preferred_element_type=jnp.float32)
    trip = jnp.dot(u22, acct, preferred_element_type=jnp.float32)
    if nv_d:
        ad = jnp.dot(vad_ref[...], a, preferred_element_type=jnp.float32)
        bd = jnp.dot(vbd_ref[...], b, preferred_element_type=jnp.float32)
        dup -= jnp.sum(jnp.minimum(ad, bd), axis=0, keepdims=True)
    if nv_t:
        at = jnp.dot(vat_ref[...], a, preferred_element_type=jnp.float32)
        bt = jnp.dot(vbt_ref[...], b, preferred_element_type=jnp.float32)
        ct = jnp.dot(vct_ref[...], c, preferred_element_type=jnp.float32)
        trip -= jnp.sum(jnp.minimum(jnp.minimum(at, bt), ct),
                        axis=0, keepdims=True)
    part = jnp.sum(dup * inv_nd + trip * inv_nt, keepdims=True)
    out_ref[...] += part


def kernel(preds, inv_d, inv_t):
    preds16 = preds.T.astype(jnp.bfloat16)
    inv_d = inv_d.astype(jnp.int32)
    inv_t = inv_t.astype(jnp.int32)
    n, ncols = preds.shape
    nd, nt = inv_d.shape[0], inv_t.shape[0]
    nv_d = _NA * _NAC - nd
    nv_t = _NA * _NAC * _NL - nt

    blk = 16384
    while n % blk:
        blk //= 2
    nsteps = n // blk
    loss = pl.pallas_call(
        functools.partial(_loss_kernel, inv_nd=1.0 / (n * nd),
                          inv_nt=1.0 / (n * nt), nv_d=nv_d, nv_t=nv_t),
        grid=(nsteps,),
        in_specs=[
            pl.BlockSpec((ncols, blk), lambda s: (0, s)),
            pl.BlockSpec(inv_d.shape, lambda s: (0, 0)),
            pl.BlockSpec(inv_t.shape, lambda s: (0, 0)),
        ],
        out_specs=pl.BlockSpec((1, 1), lambda s: (0, 0)),
        out_shape=jax.ShapeDtypeStruct((1, 1), jnp.float32),
        scratch_shapes=[pltpu.VMEM((max(nv_d, 1), _NA), jnp.bfloat16),
                        pltpu.VMEM((max(nv_d, 1), _NAC_P), jnp.bfloat16),
                        pltpu.VMEM((max(nv_t, 1), _NA), jnp.bfloat16),
                        pltpu.VMEM((max(nv_t, 1), _NAC_P), jnp.bfloat16),
                        pltpu.VMEM((max(nv_t, 1), _NL), jnp.bfloat16),
                        pltpu.VMEM((1, _NAC_P), jnp.bfloat16)],
    )(preds16, inv_d, inv_t)
    return loss.reshape(1)
